# baseline (device time: 25028 ns/iter reference)
import jax
import jax.numpy as jnp
from jax import lax
from jax.experimental import pallas as pl
from jax.experimental.pallas import tpu as pltpu

N_DEV = 4
B, SQ, DM = 2, 128, 512
HQ_LOC, DH = 4, 64
HD_LOC = HQ_LOC * DH
BLK = 64


def kernel(x, Wq, K_ext, V_ext, Wo):
    my = lax.axis_index("i")
    wq_loc = lax.dynamic_slice(Wq, (0, my * HD_LOC), (DM, HD_LOC))
    wo_loc = lax.dynamic_slice(Wo, (my * HD_LOC, 0), (HD_LOC, DM))
    k_t = jnp.transpose(K_ext, (0, 2, 1, 3))
    v_t = jnp.transpose(V_ext, (0, 2, 1, 3))

    def body(x_ref, wq_ref, k_ref, v_ref, wo_ref, out_ref,
             ctx_ref, comm_ref, send_sems, recv_sems):
        my_pos = lax.axis_index("i")
        p1 = jnp.bitwise_xor(my_pos, 1)
        p2 = (N_DEV - 1) - my_pos

        barrier = pltpu.get_barrier_semaphore()
        for nbr in (p1, p2):
            pl.semaphore_signal(barrier, inc=1, device_id=(nbr,),
                                device_id_type=pl.DeviceIdType.MESH)
        pl.semaphore_wait(barrier, 2)

        for b in range(B):
            q = jnp.dot(x_ref[b], wq_ref[...],
                        preferred_element_type=jnp.float32)
            for h in range(HQ_LOC):
                kh = k_ref[b, h]
                vh = v_ref[b, h]
                for blk in range(SQ // BLK):
                    rows = slice(blk * BLK, (blk + 1) * BLK)
                    cols = slice(h * DH, (h + 1) * DH)
                    qs = q[rows, cols]
                    ks = kh[rows]
                    vs = vh[rows]
                    s = lax.dot_general(
                        qs, ks, (((1,), (1,)), ((), ())),
                        preferred_element_type=jnp.float32) * 0.125
                    m = jnp.max(s, axis=1, keepdims=True)
                    w = jnp.exp(s - m)
                    w = w / jnp.sum(w, axis=1, keepdims=True)
                    ctx_ref[b, rows, cols] = jnp.dot(
                        w, vs, preferred_element_type=jnp.float32)
            out_ref[b] = jnp.dot(ctx_ref[b], wo_ref[...],
                                 preferred_element_type=jnp.float32)

        for p in range(2):
            partner = p1 if p == 0 else p2
            rdma = pltpu.make_async_remote_copy(
                src_ref=out_ref,
                dst_ref=comm_ref.at[p],
                send_sem=send_sems.at[p],
                recv_sem=recv_sems.at[p],
                device_id=(partner,),
                device_id_type=pl.DeviceIdType.MESH,
            )
            rdma.start()
            rdma.wait()
            out_ref[...] = out_ref[...] + comm_ref[p]

    return pl.pallas_call(
        body,
        out_shape=jax.ShapeDtypeStruct((B, SQ, DM), jnp.float32),
        in_specs=[pl.BlockSpec(memory_space=pltpu.VMEM)] * 5,
        out_specs=pl.BlockSpec(memory_space=pltpu.VMEM),
        scratch_shapes=[
            pltpu.VMEM((B, SQ, HD_LOC), jnp.float32),
            pltpu.VMEM((2, B, SQ, DM), jnp.float32),
            pltpu.SemaphoreType.DMA((2,)),
            pltpu.SemaphoreType.DMA((2,)),
        ],
        compiler_params=pltpu.CompilerParams(collective_id=0),
    )(x, wq_loc, k_t, v_t, wo_loc)


# device time: 19197 ns/iter; 1.3037x vs baseline; 1.3037x over previous
import jax
import jax.numpy as jnp
from jax import lax
from jax.experimental import pallas as pl
from jax.experimental.pallas import tpu as pltpu

N_DEV = 4
B, SQ, DM = 2, 128, 512
HQ_LOC, DH = 4, 64
HD_LOC = HQ_LOC * DH
BLK = 64
N_CHUNK = B * (SQ // BLK)


def kernel(x, Wq, K_ext, V_ext, Wo):
    my = lax.axis_index("i")
    wq_loc = lax.dynamic_slice(Wq, (0, my * HD_LOC), (DM, HD_LOC))
    wo_loc = lax.dynamic_slice(Wo, (my * HD_LOC, 0), (HD_LOC, DM))
    k_t = jnp.transpose(K_ext, (0, 2, 1, 3))
    v_t = jnp.transpose(V_ext, (0, 2, 1, 3))

    def body(x_ref, wq_ref, k_ref, v_ref, wo_ref, out_ref,
             comm_ref, send_sems, recv_sems):
        my_pos = lax.axis_index("i")
        p1 = jnp.bitwise_xor(my_pos, 1)
        p2 = (N_DEV - 1) - my_pos

        barrier = pltpu.get_barrier_semaphore()
        for nbr in (p1, p2):
            pl.semaphore_signal(barrier, inc=1, device_id=(nbr,),
                                device_id_type=pl.DeviceIdType.MESH)
        pl.semaphore_wait(barrier, 2)

        def make_rdma(phase, idx, b, blk, partner):
            return pltpu.make_async_remote_copy(
                src_ref=out_ref.at[b, pl.ds(blk * BLK, BLK), :],
                dst_ref=comm_ref.at[phase, idx],
                send_sem=send_sems.at[phase, idx],
                recv_sem=recv_sems.at[phase, idx],
                device_id=(partner,),
                device_id_type=pl.DeviceIdType.MESH,
            )

        rdma1 = []
        for b in range(B):
            q = jnp.dot(x_ref[b], wq_ref[...],
                        preferred_element_type=jnp.float32)
            for blk in range(SQ // BLK):
                rows = slice(blk * BLK, (blk + 1) * BLK)
                ctx_parts = []
                for h in range(HQ_LOC):
                    cols = slice(h * DH, (h + 1) * DH)
                    kh = k_ref[b, h]
                    vh = v_ref[b, h]
                    s = lax.dot_general(
                        q[rows, cols], kh[rows], (((1,), (1,)), ((), ())),
                        preferred_element_type=jnp.float32) * 0.125
                    m = jnp.max(s, axis=1, keepdims=True)
                    w = jnp.exp(s - m)
                    w = w / jnp.sum(w, axis=1, keepdims=True)
                    ctx_parts.append(jnp.dot(
                        w, vh[rows], preferred_element_type=jnp.float32))
                ctx_blk = jnp.concatenate(ctx_parts, axis=1)
                out_ref[b, rows, :] = jnp.dot(
                    ctx_blk, wo_ref[...],
                    preferred_element_type=jnp.float32)
                r = make_rdma(0, b * 2 + blk, b, blk, p1)
                r.start()
                rdma1.append(r)

        rdma2 = []
        for idx in range(N_CHUNK):
            b, blk = divmod(idx, 2)
            rows = slice(blk * BLK, (blk + 1) * BLK)
            rdma1[idx].wait()
            out_ref[b, rows, :] = out_ref[b, rows, :] + comm_ref[0, idx]
            r = make_rdma(1, idx, b, blk, p2)
            r.start()
            rdma2.append(r)

        for idx in range(N_CHUNK):
            b, blk = divmod(idx, 2)
            rows = slice(blk * BLK, (blk + 1) * BLK)
            rdma2[idx].wait()
            out_ref[b, rows, :] = out_ref[b, rows, :] + comm_ref[1, idx]

    return pl.pallas_call(
        body,
        out_shape=jax.ShapeDtypeStruct((B, SQ, DM), jnp.float32),
        in_specs=[pl.BlockSpec(memory_space=pltpu.VMEM)] * 5,
        out_specs=pl.BlockSpec(memory_space=pltpu.VMEM),
        scratch_shapes=[
            pltpu.VMEM((2, N_CHUNK, BLK, DM), jnp.float32),
            pltpu.SemaphoreType.DMA((2, N_CHUNK)),
            pltpu.SemaphoreType.DMA((2, N_CHUNK)),
        ],
        compiler_params=pltpu.CompilerParams(collective_id=0),
    )(x, wq_loc, k_t, v_t, wo_loc)


# device time: 17381 ns/iter; 1.4400x vs baseline; 1.1045x over previous
import jax
import jax.numpy as jnp
from jax import lax
from jax.experimental import pallas as pl
from jax.experimental.pallas import tpu as pltpu

N_DEV = 4
B, SQ, DM = 2, 128, 512
HQ_LOC, DH = 4, 64
HD_LOC = HQ_LOC * DH
BLK = 64
N_CHUNK = B * (SQ // BLK)


def kernel(x, Wq, K_ext, V_ext, Wo):
    my = lax.axis_index("i")
    wq_loc = lax.dynamic_slice(Wq, (0, my * HD_LOC), (DM, HD_LOC)) * 0.125
    wo_loc = lax.dynamic_slice(Wo, (my * HD_LOC, 0), (HD_LOC, DM))
    k_t = jnp.transpose(K_ext, (0, 2, 1, 3))
    v_t = jnp.transpose(V_ext, (0, 2, 1, 3))

    def body(x_ref, wq_ref, k_ref, v_ref, wo_ref, out_ref,
             comm_ref, send_sems, recv_sems):
        my_pos = lax.axis_index("i")
        p1 = jnp.bitwise_xor(my_pos, 1)
        p2 = (N_DEV - 1) - my_pos

        barrier = pltpu.get_barrier_semaphore()
        for nbr in (p1, p2):
            pl.semaphore_signal(barrier, inc=1, device_id=(nbr,),
                                device_id_type=pl.DeviceIdType.MESH)
        pl.semaphore_wait(barrier, 2)

        def make_rdma(phase, idx, b, blk, partner):
            return pltpu.make_async_remote_copy(
                src_ref=out_ref.at[b, pl.ds(blk * BLK, BLK), :],
                dst_ref=comm_ref.at[phase, idx],
                send_sem=send_sems.at[phase, idx],
                recv_sem=recv_sems.at[phase, idx],
                device_id=(partner,),
                device_id_type=pl.DeviceIdType.MESH,
            )

        r = lax.broadcasted_iota(jnp.int32, (SQ, SQ), 0)
        c = lax.broadcasted_iota(jnp.int32, (SQ, SQ), 1)
        maskf = (r // BLK == c // BLK).astype(jnp.float32)

        rdma1 = []
        for b in range(B):
            q = jnp.dot(x_ref[b], wq_ref[...],
                        preferred_element_type=jnp.float32)
            ctx_parts = []
            for h in range(HQ_LOC):
                s = lax.dot_general(
                    q[:, h * DH:(h + 1) * DH], k_ref[b, h],
                    (((1,), (1,)), ((), ())),
                    preferred_element_type=jnp.float32)
                w = jnp.exp(s) * maskf
                w = w / jnp.sum(w, axis=1, keepdims=True)
                ctx_parts.append(jnp.dot(
                    w, v_ref[b, h], preferred_element_type=jnp.float32))
            ctx = jnp.concatenate(ctx_parts, axis=1)
            out_ref[b] = jnp.dot(ctx, wo_ref[...],
                                 preferred_element_type=jnp.float32)
            for blk in range(SQ // BLK):
                rr = make_rdma(0, b * 2 + blk, b, blk, p1)
                rr.start()
                rdma1.append(rr)

        rdma2 = []
        for idx in range(N_CHUNK):
            b, blk = divmod(idx, 2)
            rows = slice(blk * BLK, (blk + 1) * BLK)
            rdma1[idx].wait()
            out_ref[b, rows, :] = out_ref[b, rows, :] + comm_ref[0, idx]
            rr = make_rdma(1, idx, b, blk, p2)
            rr.start()
            rdma2.append(rr)

        for idx in range(N_CHUNK):
            b, blk = divmod(idx, 2)
            rows = slice(blk * BLK, (blk + 1) * BLK)
            rdma2[idx].wait()
            out_ref[b, rows, :] = out_ref[b, rows, :] + comm_ref[1, idx]

    return pl.pallas_call(
        body,
        out_shape=jax.ShapeDtypeStruct((B, SQ, DM), jnp.float32),
        in_specs=[pl.BlockSpec(memory_space=pltpu.VMEM)] * 5,
        out_specs=pl.BlockSpec(memory_space=pltpu.VMEM),
        scratch_shapes=[
            pltpu.VMEM((2, N_CHUNK, BLK, DM), jnp.float32),
            pltpu.SemaphoreType.DMA((2, N_CHUNK)),
            pltpu.SemaphoreType.DMA((2, N_CHUNK)),
        ],
        compiler_params=pltpu.CompilerParams(collective_id=0),
    )(x, wq_loc, k_t, v_t, wo_loc)


# device time: 14112 ns/iter; 1.7735x vs baseline; 1.2316x over previous
import jax
import jax.numpy as jnp
from jax import lax
from jax.experimental import pallas as pl
from jax.experimental.pallas import tpu as pltpu

N_DEV = 4
B, SQ, DM = 2, 128, 512
HQ_LOC, DH = 4, 64
HD_LOC = HQ_LOC * DH
BLK = 64
N_PEER = 3


def kernel(x, Wq, K_ext, V_ext, Wo):
    my = lax.axis_index("i")
    wq_loc = lax.dynamic_slice(Wq, (0, my * HD_LOC), (DM, HD_LOC)) * 0.125
    wo_bf = Wo.astype(jnp.bfloat16)
    k_t = jnp.transpose(K_ext, (0, 2, 1, 3))
    v_t = jnp.transpose(V_ext, (0, 2, 1, 3))

    def body(x_ref, wq_ref, k_ref, v_ref, wo_ref, out_ref,
             ctx_ref, comm_ref, send_sems, recv_sems):
        my_pos = lax.axis_index("i")
        peers = [
            jnp.bitwise_xor(my_pos, 1),
            (N_DEV - 1) - my_pos,
            jnp.bitwise_xor(my_pos, 2),
        ]

        barrier = pltpu.get_barrier_semaphore()
        for nbr in peers:
            pl.semaphore_signal(barrier, inc=1, device_id=(nbr,),
                                device_id_type=pl.DeviceIdType.MESH)
        pl.semaphore_wait(barrier, N_PEER)

        def make_rdma(r, b, partner):
            return pltpu.make_async_remote_copy(
                src_ref=ctx_ref.at[b],
                dst_ref=comm_ref.at[r, b],
                send_sem=send_sems.at[r, b],
                recv_sem=recv_sems.at[r, b],
                device_id=(partner,),
                device_id_type=pl.DeviceIdType.MESH,
            )

        rr_ = lax.broadcasted_iota(jnp.int32, (SQ, SQ), 0)
        cc_ = lax.broadcasted_iota(jnp.int32, (SQ, SQ), 1)
        maskf = (rr_ // BLK == cc_ // BLK).astype(jnp.float32)

        sends = []
        for b in range(B):
            q = jnp.dot(x_ref[b], wq_ref[...],
                        preferred_element_type=jnp.float32)
            ctx_parts = []
            for h in range(HQ_LOC):
                s = lax.dot_general(
                    q[:, h * DH:(h + 1) * DH], k_ref[b, h],
                    (((1,), (1,)), ((), ())),
                    preferred_element_type=jnp.float32)
                w = jnp.exp(s) * maskf
                w = w / jnp.sum(w, axis=1, keepdims=True)
                ctx_parts.append(jnp.dot(
                    w, v_ref[b, h], preferred_element_type=jnp.float32))
            ctx_ref[b] = jnp.concatenate(
                ctx_parts, axis=1).astype(jnp.bfloat16)
            for r in range(N_PEER):
                rd = make_rdma(r, b, peers[r])
                rd.start()
                sends.append(rd)

        my_wo = wo_ref[pl.ds(my_pos * HD_LOC, HD_LOC), :]
        for b in range(B):
            out_ref[b] = jnp.dot(ctx_ref[b], my_wo,
                                 preferred_element_type=jnp.float32)

        for r, b in [(0, 0), (1, 0), (0, 1), (1, 1), (2, 0), (2, 1)]:
            make_rdma(r, b, peers[r]).wait_recv()
            peer_wo = wo_ref[pl.ds(peers[r] * HD_LOC, HD_LOC), :]
            out_ref[b] = out_ref[b] + jnp.dot(
                comm_ref[r, b], peer_wo,
                preferred_element_type=jnp.float32)

        for rd in sends:
            rd.wait_send()

    return pl.pallas_call(
        body,
        out_shape=jax.ShapeDtypeStruct((B, SQ, DM), jnp.float32),
        in_specs=[pl.BlockSpec(memory_space=pltpu.VMEM)] * 5,
        out_specs=pl.BlockSpec(memory_space=pltpu.VMEM),
        scratch_shapes=[
            pltpu.VMEM((B, SQ, HD_LOC), jnp.bfloat16),
            pltpu.VMEM((N_PEER, B, SQ, HD_LOC), jnp.bfloat16),
            pltpu.SemaphoreType.DMA((N_PEER, B)),
            pltpu.SemaphoreType.DMA((N_PEER, B)),
        ],
        compiler_params=pltpu.CompilerParams(collective_id=0),
    )(x, wq_loc, k_t, v_t, wo_bf)
